# baseline (speedup 1.0000x reference)
import jax
import jax.numpy as jnp
import numpy as np
from jax import lax
from jax.experimental import pallas as pl
from jax.experimental.pallas import tpu as pltpu
from jax.experimental.pallas import tpu_sc as plsc

B = 16384
I = 100
IP = 112
V = 31
E = 64
NC = 2
NS = 16
EH = E // NC
TROWS = I * V + 4
RPT = B // NS
RC = 64
NCHUNK = RPT // RC
NBUF = 4

_HIMASK = np.int32(-65536)


def _sc_kernel(x_hbm, tab_hbm, out_hbm, tab_sh, x_v, gbuf, out_v,
               sem0, sem1, sem2, sem3):
    sems = (sem0, sem1, sem2, sem3)
    c = lax.axis_index("c")
    s = lax.axis_index("s")

    @pl.when(s == 0)
    def _():
        pltpu.sync_copy(tab_hbm.at[c], tab_sh)

    plsc.subcore_barrier()

    def start(b, r):
        pltpu.async_copy(tab_sh.at[x_v.at[r]], gbuf.at[b], sems[b])

    def wait(b, r):
        pltpu.make_async_copy(
            tab_sh.at[x_v.at[r]], gbuf.at[b], sems[b]
        ).wait()

    def compute(b, r):
        z = jnp.zeros((16,), jnp.float32)
        acc = [z, z, z, z]
        for f in range(0, IP, 2):
            ps = gbuf[b, f] + gbuf[b, f + 1]
            w = plsc.bitcast(ps, jnp.int32)
            lo = plsc.bitcast(lax.shift_left(w, 16), jnp.float32)
            hi = plsc.bitcast(lax.bitwise_and(w, _HIMASK), jnp.float32)
            p = (f >> 1) & 1
            acc[p] = acc[p] + lo
            acc[2 + p] = acc[2 + p] + hi
        out_v[r, pl.ds(0, 16)] = acc[0] + acc[1]
        out_v[r, pl.ds(16, 16)] = acc[2] + acc[3]

    def chunk_body(k, _):
        base = s * RPT + k * RC
        pltpu.sync_copy(x_hbm.at[pl.ds(base, RC), :], x_v)

        for b in range(NBUF):
            start(b, b)

        def ring(g, _):
            for b in range(NBUF):
                r = g * NBUF + b
                wait(b, r)
                compute(b, r)
                start(b, r + NBUF)
            return 0

        lax.fori_loop(0, RC // NBUF - 1, ring, 0)

        for b in range(NBUF):
            r = RC - NBUF + b
            wait(b, r)
            compute(b, r)

        pltpu.sync_copy(out_v, out_hbm.at[c, pl.ds(base, RC), :])
        return 0

    lax.fori_loop(0, NCHUNK, chunk_body, 0)


@jax.jit
def _run(x_flat, tab2):
    mesh = plsc.VectorSubcoreMesh(core_axis_name="c", subcore_axis_name="s")
    f = pl.kernel(
        _sc_kernel,
        out_type=jax.ShapeDtypeStruct((NC, B, EH), jnp.float32),
        mesh=mesh,
        scratch_types=[
            pltpu.VMEM_SHARED((TROWS, EH), jnp.bfloat16),
            pltpu.VMEM((RC, IP), jnp.int32),
            pltpu.VMEM((NBUF, IP, EH), jnp.bfloat16),
            pltpu.VMEM((RC, EH), jnp.float32),
            pltpu.SemaphoreType.DMA,
            pltpu.SemaphoreType.DMA,
            pltpu.SemaphoreType.DMA,
            pltpu.SemaphoreType.DMA,
        ],
        compiler_params=pltpu.CompilerParams(
            use_tc_tiling_on_sc=False, needs_layout_passes=False
        ),
    )
    return f(x_flat, tab2)


_PERM = np.empty((EH,), np.int32)
_PERM[0::2] = np.arange(16)
_PERM[1::2] = np.arange(16) + 16


def kernel(x, embed_matrix):
    x = x.astype(jnp.int32)
    x_flat = x + (jnp.arange(I, dtype=jnp.int32) * V)[None, :]
    x_flat = jnp.concatenate(
        [x_flat, jnp.full((B, IP - I), I * V, jnp.int32)], axis=1
    )
    flat = embed_matrix.reshape(I * V, E)
    flat = jnp.concatenate(
        [flat, jnp.zeros((TROWS - I * V, E), jnp.float32)], axis=0
    )
    tab2 = flat.reshape(TROWS, NC, EH).transpose(1, 0, 2)
    tab2 = tab2[:, :, _PERM].astype(jnp.bfloat16)
    out3 = _run(x_flat, tab2)
    return out3.transpose(1, 0, 2).reshape(B, E)
